# baseline (speedup 1.0000x reference)
import functools

import jax
import jax.numpy as jnp
from jax import lax
from jax.experimental import pallas as pl
from jax.experimental.pallas import tpu as pltpu
from jax.experimental.pallas import tpu_sc as plsc

T, D, I, E = 2048, 1024, 512, 8
TILE = 128
NR_R = 5120
NR = NR_R + T
NT_R = NR_R // TILE
NT = NT_R + T // TILE


def _shift_down(y, s):
    z = jnp.zeros((s, y.shape[1]), y.dtype)
    return jnp.concatenate([z, y[:-s]], axis=0)


def _cumsum_rows(y):
    n = y.shape[0]
    s = 1
    while s < n:
        y = y + _shift_down(y, s)
        s *= 2
    return y


def _router_kernel(x_ref, wr_ref, br_ref, pos0_ref, pos1_ref, w0_ref, w1_ref,
                   eid_ref, valid_ref):
    x = x_ref[...]
    logits = jnp.dot(x, wr_ref[...], preferred_element_type=jnp.float32)
    logits = logits + br_ref[...]
    a = jax.nn.sigmoid(logits)
    idx = lax.broadcasted_iota(jnp.int32, (T, E), 1)
    m1 = jnp.max(a, axis=1, keepdims=True)
    i1 = jnp.min(jnp.where(a == m1, idx, E), axis=1, keepdims=True)
    a2 = jnp.where(idx == i1, -1.0, a)
    m2 = jnp.max(a2, axis=1, keepdims=True)
    i2 = jnp.min(jnp.where(a2 == m2, idx, E), axis=1, keepdims=True)
    denom = m1 + m2 + 1e-9
    w0_ref[...] = jnp.broadcast_to(m1 / denom, (T, 128))
    w1_ref[...] = jnp.broadcast_to(m2 / denom, (T, 128))

    oh0 = (idx == i1).astype(jnp.int32)
    oh1 = (idx == i2).astype(jnp.int32)
    cum0 = _cumsum_rows(oh0)
    cum1 = _cumsum_rows(oh1)
    cnt0 = cum0[T - 1:T, :]
    cnt1 = cum1[T - 1:T, :]
    cnt = cnt0 + cnt1
    pad_cnt = ((cnt + (TILE - 1)) >> 7) << 7

    ex = jnp.concatenate([jnp.zeros((1, 1), jnp.int32), pad_cnt[:, :E - 1]], axis=1)
    s = 1
    while s < E:
        ex = ex + jnp.concatenate([jnp.zeros((1, s), jnp.int32), ex[:, :E - s]], axis=1)
        s *= 2
    offset = ex
    total_padded = offset[:, E - 1:E] + pad_cnt[:, E - 1:E]

    rank0 = jnp.sum(cum0 * oh0, axis=1, keepdims=True) - 1
    rank1 = (jnp.sum(cnt0 * oh1, axis=1, keepdims=True)
             + jnp.sum(cum1 * oh1, axis=1, keepdims=True) - 1)
    off0 = jnp.sum(offset * oh0, axis=1, keepdims=True)
    off1 = jnp.sum(offset * oh1, axis=1, keepdims=True)
    pos0_ref[...] = off0 + rank0
    pos1_ref[...] = off1 + rank1

    rowstart = lax.broadcasted_iota(jnp.int32, (NT_R, E), 0) * TILE
    off_b = jnp.broadcast_to(offset, (NT_R, E))
    n_le = jnp.sum((rowstart >= off_b).astype(jnp.int32), axis=1, keepdims=True)
    eid_r = n_le - 1
    valid_r = (rowstart[:, :1] < total_padded).astype(jnp.int32)
    eid_ref[...] = jnp.clip(eid_r, 0, E - 1)
    valid_ref[...] = valid_r


def _run_router(x, Wr, br):
    return pl.pallas_call(
        _router_kernel,
        out_shape=[
            jax.ShapeDtypeStruct((T, 1), jnp.int32),
            jax.ShapeDtypeStruct((T, 1), jnp.int32),
            jax.ShapeDtypeStruct((T, 128), jnp.float32),
            jax.ShapeDtypeStruct((T, 128), jnp.float32),
            jax.ShapeDtypeStruct((NT_R, 1), jnp.int32),
            jax.ShapeDtypeStruct((NT_R, 1), jnp.int32),
        ],
    )(x, Wr, br.reshape(1, E))


def _silu(x):
    return x * jax.nn.sigmoid(x)


def _ffn_body(xb, wg, bg, wu, bu, wd, bd, rw):
    g = jnp.dot(xb, wg, preferred_element_type=jnp.float32) + bg
    u = jnp.dot(xb, wu, preferred_element_type=jnp.float32) + bu
    h = g * jax.nn.sigmoid(g) * u
    y = jnp.dot(h, wd, preferred_element_type=jnp.float32) + bd
    return y * rw


def _ffn_kernel(eid_s, valid_s, xs_ref, weg_ref, weu_ref, wed_ref,
                beg_ref, beu_ref, bed_ref, rw_ref, ye_ref):
    i = pl.program_id(0)

    @pl.when(valid_s[i] == 1)
    def _():
        rw = rw_ref[:, :1]
        ye_ref[...] = _ffn_body(xs_ref[...], weg_ref[0], beg_ref[0],
                                weu_ref[0], beu_ref[0], wed_ref[0],
                                bed_ref[0], rw)


def _shared_ffn_kernel(x_ref, wsg_ref, wsu_ref, wsd_ref, bsg_ref, bsu_ref,
                       bsd_ref, ysh_ref):
    ones = jnp.ones((TILE, 1), jnp.float32)
    ysh_ref[...] = _ffn_body(x_ref[...], wsg_ref[...], bsg_ref[...],
                             wsu_ref[...], bsu_ref[...], wsd_ref[...],
                             bsd_ref[...], ones)


def _run_shared_ffn(x, Wsg, bsg, Wsu, bsu, Wsd, bsd):
    return pl.pallas_call(
        _shared_ffn_kernel,
        grid=(T // TILE,),
        in_specs=[
            pl.BlockSpec((TILE, D), lambda i: (i, 0)),
            pl.BlockSpec((D, I), lambda i: (0, 0)),
            pl.BlockSpec((D, I), lambda i: (0, 0)),
            pl.BlockSpec((I, D), lambda i: (0, 0)),
            pl.BlockSpec((1, I), lambda i: (0, 0)),
            pl.BlockSpec((1, I), lambda i: (0, 0)),
            pl.BlockSpec((1, D), lambda i: (0, 0)),
        ],
        out_specs=pl.BlockSpec((TILE, D), lambda i: (i, 0)),
        out_shape=jax.ShapeDtypeStruct((T, D), jnp.float32),
    )(x, Wsg, Wsu, Wsd, bsg.reshape(1, I), bsu.reshape(1, I),
      bsd.reshape(1, D))


def _run_ffn(xs, row_w, tile_eid, tile_valid,
             Weg, beg, Weu, beu, Wed, bed):
    def emap(i, eid, val):
        return (eid[i], 0, 0)

    grid_spec = pltpu.PrefetchScalarGridSpec(
        num_scalar_prefetch=2,
        grid=(NT_R,),
        in_specs=[
            pl.BlockSpec((TILE, D), lambda i, eid, val: (i, 0)),
            pl.BlockSpec((1, D, I), emap),
            pl.BlockSpec((1, D, I), emap),
            pl.BlockSpec((1, I, D), emap),
            pl.BlockSpec((1, 1, I), emap),
            pl.BlockSpec((1, 1, I), emap),
            pl.BlockSpec((1, 1, D), emap),
            pl.BlockSpec((TILE, 128), lambda i, eid, val: (i, 0)),
        ],
        out_specs=pl.BlockSpec((TILE, D), lambda i, eid, val: (i, 0)),
    )
    return pl.pallas_call(
        _ffn_kernel,
        grid_spec=grid_spec,
        out_shape=jax.ShapeDtypeStruct((NR_R, D), jnp.float32),
    )(tile_eid, tile_valid, xs, Weg, Weu, Wed,
      beg.reshape(E, 1, I), beu.reshape(E, 1, I), bed.reshape(E, 1, D),
      row_w)


_NW = 32


def _sc_mesh():
    return plsc.VectorSubcoreMesh(core_axis_name="c", subcore_axis_name="s")


def _wid():
    return lax.axis_index("s") * 2 + lax.axis_index("c")


def _run_dispatch(x, pos0, pos1, w0r, w1r):
    toks_per_w = T // _NW

    @functools.partial(
        pl.kernel,
        out_type=[
            jax.ShapeDtypeStruct((NR_R, D), jnp.float32),
            jax.ShapeDtypeStruct((NR_R, 128), jnp.float32),
        ],
        mesh=_sc_mesh(),
        scratch_types=[
            pltpu.VMEM((toks_per_w, D), jnp.float32),
            pltpu.VMEM((toks_per_w, 128), jnp.float32),
            pltpu.VMEM((toks_per_w,), jnp.int32),
            pltpu.VMEM((toks_per_w,), jnp.int32),
            pltpu.SemaphoreType.DMA,
        ],
    )
    def k(x_hbm, p0_hbm, p1_hbm, w0_hbm, w1_hbm, xs_hbm, rw_hbm,
          xrows_v, wrep_v, i0_v, i1_v, sem):
        tb = _wid() * toks_per_w
        sl = pl.ds(tb, toks_per_w)
        pltpu.sync_copy(p0_hbm.at[sl], i0_v)
        pltpu.sync_copy(p1_hbm.at[sl], i1_v)

        pltpu.sync_copy(x_hbm.at[sl], xrows_v)
        pltpu.async_copy(xrows_v, xs_hbm.at[i0_v], sem).wait()
        pltpu.async_copy(xrows_v, xs_hbm.at[i1_v], sem).wait()

        pltpu.sync_copy(w0_hbm.at[sl], wrep_v)
        pltpu.async_copy(wrep_v, rw_hbm.at[i0_v], sem).wait()
        pltpu.sync_copy(w1_hbm.at[sl], wrep_v)
        pltpu.async_copy(wrep_v, rw_hbm.at[i1_v], sem).wait()

    return k(x, pos0, pos1, w0r, w1r)


def _run_combine(ye, ysh, pos0, pos1):
    toks_per_w = T // _NW
    CH = 8
    NCH = toks_per_w // CH

    def gather_chunk(ye_hbm, ysh_hbm, p0_hbm, p1_hbm, tb, i0, i1, r0, r1, rs, sem):
        pltpu.sync_copy(p0_hbm.at[pl.ds(tb, CH)], i0)
        pltpu.sync_copy(p1_hbm.at[pl.ds(tb, CH)], i1)
        pltpu.async_copy(ye_hbm.at[i0], r0, sem)
        pltpu.async_copy(ye_hbm.at[i1], r1, sem)
        pltpu.async_copy(ysh_hbm.at[pl.ds(tb, CH)], rs, sem)

    def wait_chunk(ye_hbm, i0, i1, r0, r1, rs, sem):
        pltpu.make_async_copy(ye_hbm.at[i0], r0, sem).wait()
        pltpu.make_async_copy(ye_hbm.at[i1], r1, sem).wait()
        pltpu.make_async_copy(ye_hbm.at[i1], rs, sem).wait()

    def compute_chunk(out_hbm, tb, r0, r1, rs, o_v):
        def tok(i, carry2):
            def grp(g, carry3):
                for q in range(4):
                    s = pl.ds((g * 4 + q) * 16, 16)
                    o_v[i, s] = r0[i, s] + r1[i, s] + rs[i, s]
                return carry3
            return lax.fori_loop(0, D // 64, grp, carry2)
        lax.fori_loop(0, CH, tok, 0)
        pltpu.sync_copy(o_v, out_hbm.at[pl.ds(tb, CH)])

    @functools.partial(
        pl.kernel,
        out_type=jax.ShapeDtypeStruct((T, D), jnp.float32),
        mesh=_sc_mesh(),
        scratch_types=[
            pltpu.VMEM((CH,), jnp.int32),
            pltpu.VMEM((CH,), jnp.int32),
            pltpu.VMEM((CH,), jnp.int32),
            pltpu.VMEM((CH,), jnp.int32),
            pltpu.VMEM((CH, D), jnp.float32),
            pltpu.VMEM((CH, D), jnp.float32),
            pltpu.VMEM((CH, D), jnp.float32),
            pltpu.VMEM((CH, D), jnp.float32),
            pltpu.VMEM((CH, D), jnp.float32),
            pltpu.VMEM((CH, D), jnp.float32),
            pltpu.VMEM((CH, D), jnp.float32),
            pltpu.SemaphoreType.DMA,
            pltpu.SemaphoreType.DMA,
        ],
    )
    def k(ye_hbm, ysh_hbm, p0_hbm, p1_hbm, out_hbm,
          i0a, i1a, i0b, i1b, r0a, r1a, rsa, r0b, r1b, rsb, o_v, semA, semB):
        base = _wid() * toks_per_w
        gather_chunk(ye_hbm, ysh_hbm, p0_hbm, p1_hbm, base,
                     i0a, i1a, r0a, r1a, rsa, semA)

        def body(c, carry):
            tb = base + c * CH

            @pl.when(c % 2 == 0)
            def _():
                @pl.when(c + 1 < NCH)
                def _():
                    gather_chunk(ye_hbm, ysh_hbm, p0_hbm, p1_hbm, tb + CH,
                                 i0b, i1b, r0b, r1b, rsb, semB)
                wait_chunk(ye_hbm, i0a, i1a, r0a, r1a, rsa, semA)
                compute_chunk(out_hbm, tb, r0a, r1a, rsa, o_v)

            @pl.when(c % 2 == 1)
            def _():
                @pl.when(c + 1 < NCH)
                def _():
                    gather_chunk(ye_hbm, ysh_hbm, p0_hbm, p1_hbm, tb + CH,
                                 i0a, i1a, r0a, r1a, rsa, semA)
                wait_chunk(ye_hbm, i0b, i1b, r0b, r1b, rsb, semB)
                compute_chunk(out_hbm, tb, r0b, r1b, rsb, o_v)

            return carry

        lax.fori_loop(0, NCH, body, 0)

    return k(ye, ysh, pos0, pos1)


def kernel(hidden_states, Wr, br, Wsg, bsg, Wsu, bsu, Wsd, bsd, Weg, beg, Weu, beu, Wed, bed):
    x = hidden_states
    pos0c, pos1c, w0r, w1r, eidc, validc = _run_router(x, Wr, br)
    pos0, pos1 = pos0c[:, 0], pos1c[:, 0]
    tile_eid, tile_valid = eidc[:, 0], validc[:, 0]

    xs, rw = _run_dispatch(x, pos0, pos1, w0r, w1r)
    ysh = _run_shared_ffn(x, Wsg, bsg, Wsu, bsu, Wsd, bsd)
    ye = _run_ffn(xs, rw, tile_eid, tile_valid,
                  Weg, beg, Weu, beu, Wed, bed)

    out = _run_combine(ye, ysh, pos0, pos1)
    return out


# speedup vs baseline: 1.0770x; 1.0770x over previous
import functools

import jax
import jax.numpy as jnp
from jax import lax
from jax.experimental import pallas as pl
from jax.experimental.pallas import tpu as pltpu
from jax.experimental.pallas import tpu_sc as plsc

T, D, I, E = 2048, 1024, 512, 8
TILE = 128
NR_R = 5120
NR = NR_R + T
NT_R = NR_R // TILE
NT = NT_R + T // TILE


def _shift_down(y, s):
    z = jnp.zeros((s, y.shape[1]), y.dtype)
    return jnp.concatenate([z, y[:-s]], axis=0)


def _cumsum_rows(y):
    n = y.shape[0]
    s = 1
    while s < n:
        y = y + _shift_down(y, s)
        s *= 2
    return y


def _router_kernel(x_ref, wr_ref, br_ref, pos0_ref, pos1_ref, w0_ref, w1_ref,
                   eid_ref, valid_ref):
    x = x_ref[...]
    logits = jnp.dot(x, wr_ref[...], preferred_element_type=jnp.float32)
    logits = logits + br_ref[...]
    a = jax.nn.sigmoid(logits)
    idx = lax.broadcasted_iota(jnp.int32, (T, E), 1)
    m1 = jnp.max(a, axis=1, keepdims=True)
    i1 = jnp.min(jnp.where(a == m1, idx, E), axis=1, keepdims=True)
    a2 = jnp.where(idx == i1, -1.0, a)
    m2 = jnp.max(a2, axis=1, keepdims=True)
    i2 = jnp.min(jnp.where(a2 == m2, idx, E), axis=1, keepdims=True)
    denom = m1 + m2 + 1e-9
    w0_ref[...] = jnp.broadcast_to(m1 / denom, (T, 128))
    w1_ref[...] = jnp.broadcast_to(m2 / denom, (T, 128))

    oh0 = (idx == i1).astype(jnp.int32)
    oh1 = (idx == i2).astype(jnp.int32)
    cum0 = _cumsum_rows(oh0)
    cum1 = _cumsum_rows(oh1)
    cnt0 = cum0[T - 1:T, :]
    cnt1 = cum1[T - 1:T, :]
    cnt = cnt0 + cnt1
    pad_cnt = ((cnt + (TILE - 1)) >> 7) << 7

    ex = jnp.concatenate([jnp.zeros((1, 1), jnp.int32), pad_cnt[:, :E - 1]], axis=1)
    s = 1
    while s < E:
        ex = ex + jnp.concatenate([jnp.zeros((1, s), jnp.int32), ex[:, :E - s]], axis=1)
        s *= 2
    offset = ex
    total_padded = offset[:, E - 1:E] + pad_cnt[:, E - 1:E]

    rank0 = jnp.sum(cum0 * oh0, axis=1, keepdims=True) - 1
    rank1 = (jnp.sum(cnt0 * oh1, axis=1, keepdims=True)
             + jnp.sum(cum1 * oh1, axis=1, keepdims=True) - 1)
    off0 = jnp.sum(offset * oh0, axis=1, keepdims=True)
    off1 = jnp.sum(offset * oh1, axis=1, keepdims=True)
    pos0_ref[...] = off0 + rank0
    pos1_ref[...] = off1 + rank1

    rowstart = lax.broadcasted_iota(jnp.int32, (NT_R, E), 0) * TILE
    off_b = jnp.broadcast_to(offset, (NT_R, E))
    n_le = jnp.sum((rowstart >= off_b).astype(jnp.int32), axis=1, keepdims=True)
    eid_r = n_le - 1
    valid_r = (rowstart[:, :1] < total_padded).astype(jnp.int32)
    eid_ref[...] = jnp.clip(eid_r, 0, E - 1)
    valid_ref[...] = valid_r


def _run_router(x, Wr, br):
    return pl.pallas_call(
        _router_kernel,
        out_shape=[
            jax.ShapeDtypeStruct((T, 1), jnp.int32),
            jax.ShapeDtypeStruct((T, 1), jnp.int32),
            jax.ShapeDtypeStruct((T, 128), jnp.float32),
            jax.ShapeDtypeStruct((T, 128), jnp.float32),
            jax.ShapeDtypeStruct((NT_R, 1), jnp.int32),
            jax.ShapeDtypeStruct((NT_R, 1), jnp.int32),
        ],
    )(x, Wr, br.reshape(1, E))


def _silu(x):
    return x * jax.nn.sigmoid(x)


def _ffn_body(xb, wg, bg, wu, bu, wd, bd, rw):
    g = jnp.dot(xb, wg, preferred_element_type=jnp.float32) + bg
    u = jnp.dot(xb, wu, preferred_element_type=jnp.float32) + bu
    h = g * jax.nn.sigmoid(g) * u
    y = jnp.dot(h, wd, preferred_element_type=jnp.float32) + bd
    return y * rw


def _ffn_kernel(eid_s, valid_s, xs_ref, weg_ref, weu_ref, wed_ref,
                beg_ref, beu_ref, bed_ref, rw_ref, ye_ref):
    i = pl.program_id(0)

    @pl.when(valid_s[i] == 1)
    def _():
        rw = rw_ref[:, :1]
        ye_ref[...] = _ffn_body(xs_ref[...], weg_ref[0], beg_ref[0],
                                weu_ref[0], beu_ref[0], wed_ref[0],
                                bed_ref[0], rw)


def _shared_ffn_kernel(x_ref, wsg_ref, wsu_ref, wsd_ref, bsg_ref, bsu_ref,
                       bsd_ref, ysh_ref):
    ones = jnp.ones((TILE, 1), jnp.float32)
    ysh_ref[...] = _ffn_body(x_ref[...], wsg_ref[...], bsg_ref[...],
                             wsu_ref[...], bsu_ref[...], wsd_ref[...],
                             bsd_ref[...], ones)


def _run_shared_ffn(x, Wsg, bsg, Wsu, bsu, Wsd, bsd):
    return pl.pallas_call(
        _shared_ffn_kernel,
        grid=(T // TILE,),
        in_specs=[
            pl.BlockSpec((TILE, D), lambda i: (i, 0)),
            pl.BlockSpec((D, I), lambda i: (0, 0)),
            pl.BlockSpec((D, I), lambda i: (0, 0)),
            pl.BlockSpec((I, D), lambda i: (0, 0)),
            pl.BlockSpec((1, I), lambda i: (0, 0)),
            pl.BlockSpec((1, I), lambda i: (0, 0)),
            pl.BlockSpec((1, D), lambda i: (0, 0)),
        ],
        out_specs=pl.BlockSpec((TILE, D), lambda i: (i, 0)),
        out_shape=jax.ShapeDtypeStruct((T, D), jnp.float32),
    )(x, Wsg, Wsu, Wsd, bsg.reshape(1, I), bsu.reshape(1, I),
      bsd.reshape(1, D))


def _run_ffn(xs, row_w, tile_eid, tile_valid,
             Weg, beg, Weu, beu, Wed, bed):
    def emap(i, eid, val):
        return (eid[i], 0, 0)

    grid_spec = pltpu.PrefetchScalarGridSpec(
        num_scalar_prefetch=2,
        grid=(NT_R,),
        in_specs=[
            pl.BlockSpec((TILE, D), lambda i, eid, val: (i, 0)),
            pl.BlockSpec((1, D, I), emap),
            pl.BlockSpec((1, D, I), emap),
            pl.BlockSpec((1, I, D), emap),
            pl.BlockSpec((1, 1, I), emap),
            pl.BlockSpec((1, 1, I), emap),
            pl.BlockSpec((1, 1, D), emap),
            pl.BlockSpec((TILE, 128), lambda i, eid, val: (i, 0)),
        ],
        out_specs=pl.BlockSpec((TILE, D), lambda i, eid, val: (i, 0)),
    )
    return pl.pallas_call(
        _ffn_kernel,
        grid_spec=grid_spec,
        out_shape=jax.ShapeDtypeStruct((NR_R, D), jnp.float32),
    )(tile_eid, tile_valid, xs, Weg, Weu, Wed,
      beg.reshape(E, 1, I), beu.reshape(E, 1, I), bed.reshape(E, 1, D),
      row_w)


_NW = 32


def _sc_mesh():
    return plsc.VectorSubcoreMesh(core_axis_name="c", subcore_axis_name="s")


def _wid():
    return lax.axis_index("s") * 2 + lax.axis_index("c")


def _run_dispatch(x, pos0, pos1, w0r, w1r):
    toks_per_w = T // _NW

    @functools.partial(
        pl.kernel,
        out_type=[
            jax.ShapeDtypeStruct((NR_R, D), jnp.float32),
            jax.ShapeDtypeStruct((NR_R, 128), jnp.float32),
        ],
        mesh=_sc_mesh(),
        scratch_types=[
            pltpu.VMEM((toks_per_w, D), jnp.float32),
            pltpu.VMEM((toks_per_w, 128), jnp.float32),
            pltpu.VMEM((toks_per_w,), jnp.int32),
            pltpu.VMEM((toks_per_w,), jnp.int32),
            pltpu.SemaphoreType.DMA,
        ],
    )
    def k(x_hbm, p0_hbm, p1_hbm, w0_hbm, w1_hbm, xs_hbm, rw_hbm,
          xrows_v, wrep_v, i0_v, i1_v, sem):
        tb = _wid() * toks_per_w
        sl = pl.ds(tb, toks_per_w)
        pltpu.sync_copy(p0_hbm.at[sl], i0_v)
        pltpu.sync_copy(p1_hbm.at[sl], i1_v)

        pltpu.sync_copy(x_hbm.at[sl], xrows_v)
        pltpu.async_copy(xrows_v, xs_hbm.at[i0_v], sem).wait()
        pltpu.async_copy(xrows_v, xs_hbm.at[i1_v], sem).wait()

        pltpu.sync_copy(w0_hbm.at[sl], wrep_v)
        pltpu.async_copy(wrep_v, rw_hbm.at[i0_v], sem).wait()
        pltpu.sync_copy(w1_hbm.at[sl], wrep_v)
        pltpu.async_copy(wrep_v, rw_hbm.at[i1_v], sem).wait()

    return k(x, pos0, pos1, w0r, w1r)


def _run_combine_gather(ye, pos0, pos1):
    toks_per_w = T // _NW
    CH = 16

    @functools.partial(
        pl.kernel,
        out_type=jax.ShapeDtypeStruct((2, T, D), jnp.float32),
        mesh=_sc_mesh(),
        scratch_types=[
            pltpu.VMEM((CH,), jnp.int32),
            pltpu.VMEM((CH,), jnp.int32),
            pltpu.VMEM((CH, D), jnp.float32),
            pltpu.VMEM((CH, D), jnp.float32),
            pltpu.SemaphoreType.DMA,
        ],
    )
    def k(ye_hbm, p0_hbm, p1_hbm, rr_hbm, i0_v, i1_v, r0_v, r1_v, sem):
        base = _wid() * toks_per_w

        def body(c, carry):
            tb = base + c * CH
            pltpu.sync_copy(p0_hbm.at[pl.ds(tb, CH)], i0_v)
            pltpu.sync_copy(p1_hbm.at[pl.ds(tb, CH)], i1_v)
            d0 = pltpu.async_copy(ye_hbm.at[i0_v], r0_v, sem)
            d1 = pltpu.async_copy(ye_hbm.at[i1_v], r1_v, sem)
            d0.wait()
            pltpu.sync_copy(r0_v, rr_hbm.at[0, pl.ds(tb, CH)])
            d1.wait()
            pltpu.sync_copy(r1_v, rr_hbm.at[1, pl.ds(tb, CH)])
            return carry

        lax.fori_loop(0, toks_per_w // CH, body, 0)

    return k(ye, pos0, pos1)


def _add3_kernel(r0_ref, r1_ref, rs_ref, out_ref):
    out_ref[...] = r0_ref[0] + r1_ref[0] + rs_ref[...]


def _run_add3(rr, ysh):
    return pl.pallas_call(
        _add3_kernel,
        grid=(T // TILE,),
        in_specs=[
            pl.BlockSpec((1, TILE, D), lambda i: (0, i, 0)),
            pl.BlockSpec((1, TILE, D), lambda i: (1, i, 0)),
            pl.BlockSpec((TILE, D), lambda i: (i, 0)),
        ],
        out_specs=pl.BlockSpec((TILE, D), lambda i: (i, 0)),
        out_shape=jax.ShapeDtypeStruct((T, D), jnp.float32),
    )(rr, rr, ysh)


def kernel(hidden_states, Wr, br, Wsg, bsg, Wsu, bsu, Wsd, bsd, Weg, beg, Weu, beu, Wed, bed):
    x = hidden_states
    pos0c, pos1c, w0r, w1r, eidc, validc = _run_router(x, Wr, br)
    pos0, pos1 = pos0c[:, 0], pos1c[:, 0]
    tile_eid, tile_valid = eidc[:, 0], validc[:, 0]

    xs, rw = _run_dispatch(x, pos0, pos1, w0r, w1r)
    ysh = _run_shared_ffn(x, Wsg, bsg, Wsu, bsu, Wsd, bsd)
    ye = _run_ffn(xs, rw, tile_eid, tile_valid,
                  Weg, beg, Weu, beu, Wed, bed)

    rr = _run_combine_gather(ye, pos0, pos1)
    out = _run_add3(rr, ysh)
    return out


# speedup vs baseline: 1.1971x; 1.1116x over previous
import functools

import jax
import jax.numpy as jnp
from jax import lax
from jax.experimental import pallas as pl
from jax.experimental.pallas import tpu as pltpu
from jax.experimental.pallas import tpu_sc as plsc

T, D, I, E = 2048, 1024, 512, 8
TILE = 256
NR_R = 6144
NR = NR_R + T
NT_R = NR_R // TILE
NT = NT_R + T // TILE


def _shift_down(y, s):
    z = jnp.zeros((s, y.shape[1]), y.dtype)
    return jnp.concatenate([z, y[:-s]], axis=0)


def _cumsum_rows(y):
    n = y.shape[0]
    s = 1
    while s < n:
        y = y + _shift_down(y, s)
        s *= 2
    return y


def _router_kernel(x_ref, wr_ref, br_ref, pos0_ref, pos1_ref, w0_ref, w1_ref,
                   eid_ref, valid_ref):
    x = x_ref[...]
    logits = jnp.dot(x, wr_ref[...], preferred_element_type=jnp.float32)
    logits = logits + br_ref[...]
    a = jax.nn.sigmoid(logits)
    idx = lax.broadcasted_iota(jnp.int32, (T, E), 1)
    m1 = jnp.max(a, axis=1, keepdims=True)
    i1 = jnp.min(jnp.where(a == m1, idx, E), axis=1, keepdims=True)
    a2 = jnp.where(idx == i1, -1.0, a)
    m2 = jnp.max(a2, axis=1, keepdims=True)
    i2 = jnp.min(jnp.where(a2 == m2, idx, E), axis=1, keepdims=True)
    denom = m1 + m2 + 1e-9
    w0_ref[...] = jnp.broadcast_to(m1 / denom, (T, 128))
    w1_ref[...] = jnp.broadcast_to(m2 / denom, (T, 128))

    oh0 = (idx == i1).astype(jnp.int32)
    oh1 = (idx == i2).astype(jnp.int32)
    cum0 = _cumsum_rows(oh0)
    cum1 = _cumsum_rows(oh1)
    cnt0 = cum0[T - 1:T, :]
    cnt1 = cum1[T - 1:T, :]
    cnt = cnt0 + cnt1
    pad_cnt = ((cnt + (TILE - 1)) >> 8) << 8

    ex = jnp.concatenate([jnp.zeros((1, 1), jnp.int32), pad_cnt[:, :E - 1]], axis=1)
    s = 1
    while s < E:
        ex = ex + jnp.concatenate([jnp.zeros((1, s), jnp.int32), ex[:, :E - s]], axis=1)
        s *= 2
    offset = ex
    total_padded = offset[:, E - 1:E] + pad_cnt[:, E - 1:E]

    rank0 = jnp.sum(cum0 * oh0, axis=1, keepdims=True) - 1
    rank1 = (jnp.sum(cnt0 * oh1, axis=1, keepdims=True)
             + jnp.sum(cum1 * oh1, axis=1, keepdims=True) - 1)
    off0 = jnp.sum(offset * oh0, axis=1, keepdims=True)
    off1 = jnp.sum(offset * oh1, axis=1, keepdims=True)
    pos0_ref[...] = off0 + rank0
    pos1_ref[...] = off1 + rank1

    rowstart = lax.broadcasted_iota(jnp.int32, (NT_R, E), 0) * TILE
    off_b = jnp.broadcast_to(offset, (NT_R, E))
    n_le = jnp.sum((rowstart >= off_b).astype(jnp.int32), axis=1, keepdims=True)
    eid_r = n_le - 1
    valid_r = (rowstart[:, :1] < total_padded).astype(jnp.int32)
    eid_ref[...] = jnp.clip(eid_r, 0, E - 1)
    valid_ref[...] = valid_r


def _run_router(x, Wr, br):
    return pl.pallas_call(
        _router_kernel,
        out_shape=[
            jax.ShapeDtypeStruct((T, 1), jnp.int32),
            jax.ShapeDtypeStruct((T, 1), jnp.int32),
            jax.ShapeDtypeStruct((T, 128), jnp.float32),
            jax.ShapeDtypeStruct((T, 128), jnp.float32),
            jax.ShapeDtypeStruct((NT_R, 1), jnp.int32),
            jax.ShapeDtypeStruct((NT_R, 1), jnp.int32),
        ],
    )(x, Wr, br.reshape(1, E))


def _silu(x):
    return x * jax.nn.sigmoid(x)


def _ffn_body(xb, wg, bg, wu, bu, wd, bd, rw):
    g = jnp.dot(xb, wg, preferred_element_type=jnp.float32) + bg
    u = jnp.dot(xb, wu, preferred_element_type=jnp.float32) + bu
    h = g * jax.nn.sigmoid(g) * u
    y = jnp.dot(h, wd, preferred_element_type=jnp.float32) + bd
    return y * rw


def _ffn_kernel(eid_s, valid_s, xs_ref, weg_ref, weu_ref, wed_ref,
                beg_ref, beu_ref, bed_ref, rw_ref, ye_ref):
    i = pl.program_id(0)

    @pl.when(valid_s[i] == 1)
    def _():
        rw = rw_ref[:, :1]
        ye_ref[...] = _ffn_body(xs_ref[...], weg_ref[0], beg_ref[0],
                                weu_ref[0], beu_ref[0], wed_ref[0],
                                bed_ref[0], rw)


def _shared_ffn_kernel(x_ref, wsg_ref, wsu_ref, wsd_ref, bsg_ref, bsu_ref,
                       bsd_ref, ysh_ref):
    ones = jnp.ones((TILE, 1), jnp.float32)
    ysh_ref[...] = _ffn_body(x_ref[...], wsg_ref[...], bsg_ref[...],
                             wsu_ref[...], bsu_ref[...], wsd_ref[...],
                             bsd_ref[...], ones)


def _run_shared_ffn(x, Wsg, bsg, Wsu, bsu, Wsd, bsd):
    return pl.pallas_call(
        _shared_ffn_kernel,
        grid=(T // TILE,),
        in_specs=[
            pl.BlockSpec((TILE, D), lambda i: (i, 0)),
            pl.BlockSpec((D, I), lambda i: (0, 0)),
            pl.BlockSpec((D, I), lambda i: (0, 0)),
            pl.BlockSpec((I, D), lambda i: (0, 0)),
            pl.BlockSpec((1, I), lambda i: (0, 0)),
            pl.BlockSpec((1, I), lambda i: (0, 0)),
            pl.BlockSpec((1, D), lambda i: (0, 0)),
        ],
        out_specs=pl.BlockSpec((TILE, D), lambda i: (i, 0)),
        out_shape=jax.ShapeDtypeStruct((T, D), jnp.float32),
    )(x, Wsg, Wsu, Wsd, bsg.reshape(1, I), bsu.reshape(1, I),
      bsd.reshape(1, D))


def _run_ffn(xs, row_w, tile_eid, tile_valid,
             Weg, beg, Weu, beu, Wed, bed):
    def emap(i, eid, val):
        return (eid[i], 0, 0)

    grid_spec = pltpu.PrefetchScalarGridSpec(
        num_scalar_prefetch=2,
        grid=(NT_R,),
        in_specs=[
            pl.BlockSpec((TILE, D), lambda i, eid, val: (i, 0)),
            pl.BlockSpec((1, D, I), emap),
            pl.BlockSpec((1, D, I), emap),
            pl.BlockSpec((1, I, D), emap),
            pl.BlockSpec((1, 1, I), emap),
            pl.BlockSpec((1, 1, I), emap),
            pl.BlockSpec((1, 1, D), emap),
            pl.BlockSpec((TILE, 128), lambda i, eid, val: (i, 0)),
        ],
        out_specs=pl.BlockSpec((TILE, D), lambda i, eid, val: (i, 0)),
    )
    return pl.pallas_call(
        _ffn_kernel,
        grid_spec=grid_spec,
        out_shape=jax.ShapeDtypeStruct((NR_R, D), jnp.float32),
    )(tile_eid, tile_valid, xs, Weg, Weu, Wed,
      beg.reshape(E, 1, I), beu.reshape(E, 1, I), bed.reshape(E, 1, D),
      row_w)


_NW = 32


def _sc_mesh():
    return plsc.VectorSubcoreMesh(core_axis_name="c", subcore_axis_name="s")


def _wid():
    return lax.axis_index("s") * 2 + lax.axis_index("c")


def _run_dispatch(x, pos0, pos1, w0r, w1r):
    toks_per_w = T // _NW

    @functools.partial(
        pl.kernel,
        out_type=[
            jax.ShapeDtypeStruct((NR_R, D), jnp.float32),
            jax.ShapeDtypeStruct((NR_R, 128), jnp.float32),
        ],
        mesh=_sc_mesh(),
        scratch_types=[
            pltpu.VMEM((toks_per_w, D), jnp.float32),
            pltpu.VMEM((toks_per_w, 128), jnp.float32),
            pltpu.VMEM((toks_per_w,), jnp.int32),
            pltpu.VMEM((toks_per_w,), jnp.int32),
            pltpu.SemaphoreType.DMA,
        ],
    )
    def k(x_hbm, p0_hbm, p1_hbm, w0_hbm, w1_hbm, xs_hbm, rw_hbm,
          xrows_v, wrep_v, i0_v, i1_v, sem):
        tb = _wid() * toks_per_w
        sl = pl.ds(tb, toks_per_w)
        pltpu.sync_copy(p0_hbm.at[sl], i0_v)
        pltpu.sync_copy(p1_hbm.at[sl], i1_v)

        pltpu.sync_copy(x_hbm.at[sl], xrows_v)
        pltpu.async_copy(xrows_v, xs_hbm.at[i0_v], sem).wait()
        pltpu.async_copy(xrows_v, xs_hbm.at[i1_v], sem).wait()

        pltpu.sync_copy(w0_hbm.at[sl], wrep_v)
        pltpu.async_copy(wrep_v, rw_hbm.at[i0_v], sem).wait()
        pltpu.sync_copy(w1_hbm.at[sl], wrep_v)
        pltpu.async_copy(wrep_v, rw_hbm.at[i1_v], sem).wait()

    return k(x, pos0, pos1, w0r, w1r)


def _run_combine_gather(ye, pos0, pos1):
    toks_per_w = T // _NW
    CH = 16

    @functools.partial(
        pl.kernel,
        out_type=jax.ShapeDtypeStruct((2, T, D), jnp.float32),
        mesh=_sc_mesh(),
        scratch_types=[
            pltpu.VMEM((CH,), jnp.int32),
            pltpu.VMEM((CH,), jnp.int32),
            pltpu.VMEM((CH, D), jnp.float32),
            pltpu.VMEM((CH, D), jnp.float32),
            pltpu.SemaphoreType.DMA,
        ],
    )
    def k(ye_hbm, p0_hbm, p1_hbm, rr_hbm, i0_v, i1_v, r0_v, r1_v, sem):
        base = _wid() * toks_per_w

        def body(c, carry):
            tb = base + c * CH
            pltpu.sync_copy(p0_hbm.at[pl.ds(tb, CH)], i0_v)
            pltpu.sync_copy(p1_hbm.at[pl.ds(tb, CH)], i1_v)
            d0 = pltpu.async_copy(ye_hbm.at[i0_v], r0_v, sem)
            d1 = pltpu.async_copy(ye_hbm.at[i1_v], r1_v, sem)
            d0.wait()
            pltpu.sync_copy(r0_v, rr_hbm.at[0, pl.ds(tb, CH)])
            d1.wait()
            pltpu.sync_copy(r1_v, rr_hbm.at[1, pl.ds(tb, CH)])
            return carry

        lax.fori_loop(0, toks_per_w // CH, body, 0)

    return k(ye, pos0, pos1)


def _add3_kernel(r0_ref, r1_ref, rs_ref, out_ref):
    out_ref[...] = r0_ref[0] + r1_ref[0] + rs_ref[...]


def _run_add3(rr, ysh):
    return pl.pallas_call(
        _add3_kernel,
        grid=(T // TILE,),
        in_specs=[
            pl.BlockSpec((1, TILE, D), lambda i: (0, i, 0)),
            pl.BlockSpec((1, TILE, D), lambda i: (1, i, 0)),
            pl.BlockSpec((TILE, D), lambda i: (i, 0)),
        ],
        out_specs=pl.BlockSpec((TILE, D), lambda i: (i, 0)),
        out_shape=jax.ShapeDtypeStruct((T, D), jnp.float32),
    )(rr, rr, ysh)


def kernel(hidden_states, Wr, br, Wsg, bsg, Wsu, bsu, Wsd, bsd, Weg, beg, Weu, beu, Wed, bed):
    x = hidden_states
    pos0c, pos1c, w0r, w1r, eidc, validc = _run_router(x, Wr, br)
    pos0, pos1 = pos0c[:, 0], pos1c[:, 0]
    tile_eid, tile_valid = eidc[:, 0], validc[:, 0]

    xs, rw = _run_dispatch(x, pos0, pos1, w0r, w1r)
    ysh = _run_shared_ffn(x, Wsg, bsg, Wsu, bsu, Wsd, bsd)
    ye = _run_ffn(xs, rw, tile_eid, tile_valid,
                  Weg, beg, Weu, beu, Wed, bed)

    rr = _run_combine_gather(ye, pos0, pos1)
    out = _run_add3(rr, ysh)
    return out


# speedup vs baseline: 1.2404x; 1.0362x over previous
import functools

import jax
import jax.numpy as jnp
from jax import lax
from jax.experimental import pallas as pl
from jax.experimental.pallas import tpu as pltpu
from jax.experimental.pallas import tpu_sc as plsc

T, D, I, E = 2048, 1024, 512, 8
TILE = 512
NR_R = 8192
NR = NR_R + T
NT_R = NR_R // TILE
NT = NT_R + T // TILE


def _shift_down(y, s):
    z = jnp.zeros((s, y.shape[1]), y.dtype)
    return jnp.concatenate([z, y[:-s]], axis=0)


def _cumsum_rows(y):
    n = y.shape[0]
    s = 1
    while s < n:
        y = y + _shift_down(y, s)
        s *= 2
    return y


def _router_kernel(x_ref, wr_ref, br_ref, pos0_ref, pos1_ref, w0_ref, w1_ref,
                   eid_ref, valid_ref):
    x = x_ref[...]
    logits = jnp.dot(x, wr_ref[...], preferred_element_type=jnp.float32)
    logits = logits + br_ref[...]
    a = jax.nn.sigmoid(logits)
    idx = lax.broadcasted_iota(jnp.int32, (T, E), 1)
    m1 = jnp.max(a, axis=1, keepdims=True)
    i1 = jnp.min(jnp.where(a == m1, idx, E), axis=1, keepdims=True)
    a2 = jnp.where(idx == i1, -1.0, a)
    m2 = jnp.max(a2, axis=1, keepdims=True)
    i2 = jnp.min(jnp.where(a2 == m2, idx, E), axis=1, keepdims=True)
    denom = m1 + m2 + 1e-9
    w0_ref[...] = jnp.broadcast_to(m1 / denom, (T, 128))
    w1_ref[...] = jnp.broadcast_to(m2 / denom, (T, 128))

    oh0 = (idx == i1).astype(jnp.int32)
    oh1 = (idx == i2).astype(jnp.int32)
    cum0 = _cumsum_rows(oh0)
    cum1 = _cumsum_rows(oh1)
    cnt0 = cum0[T - 1:T, :]
    cnt1 = cum1[T - 1:T, :]
    cnt = cnt0 + cnt1
    pad_cnt = ((cnt + (TILE - 1)) >> 9) << 9

    ex = jnp.concatenate([jnp.zeros((1, 1), jnp.int32), pad_cnt[:, :E - 1]], axis=1)
    s = 1
    while s < E:
        ex = ex + jnp.concatenate([jnp.zeros((1, s), jnp.int32), ex[:, :E - s]], axis=1)
        s *= 2
    offset = ex
    total_padded = offset[:, E - 1:E] + pad_cnt[:, E - 1:E]

    rank0 = jnp.sum(cum0 * oh0, axis=1, keepdims=True) - 1
    rank1 = (jnp.sum(cnt0 * oh1, axis=1, keepdims=True)
             + jnp.sum(cum1 * oh1, axis=1, keepdims=True) - 1)
    off0 = jnp.sum(offset * oh0, axis=1, keepdims=True)
    off1 = jnp.sum(offset * oh1, axis=1, keepdims=True)
    pos0_ref[...] = off0 + rank0
    pos1_ref[...] = off1 + rank1

    rowstart = lax.broadcasted_iota(jnp.int32, (NT_R, E), 0) * TILE
    off_b = jnp.broadcast_to(offset, (NT_R, E))
    n_le = jnp.sum((rowstart >= off_b).astype(jnp.int32), axis=1, keepdims=True)
    eid_r = n_le - 1
    valid_r = (rowstart[:, :1] < total_padded).astype(jnp.int32)
    eid_ref[...] = jnp.clip(eid_r, 0, E - 1)
    valid_ref[...] = valid_r


def _run_router(x, Wr, br):
    return pl.pallas_call(
        _router_kernel,
        out_shape=[
            jax.ShapeDtypeStruct((T, 1), jnp.int32),
            jax.ShapeDtypeStruct((T, 1), jnp.int32),
            jax.ShapeDtypeStruct((T, 128), jnp.float32),
            jax.ShapeDtypeStruct((T, 128), jnp.float32),
            jax.ShapeDtypeStruct((NT_R, 1), jnp.int32),
            jax.ShapeDtypeStruct((NT_R, 1), jnp.int32),
        ],
    )(x, Wr, br.reshape(1, E))


def _silu(x):
    return x * jax.nn.sigmoid(x)


def _ffn_body(xb, wg, bg, wu, bu, wd, bd, rw):
    g = jnp.dot(xb, wg, preferred_element_type=jnp.float32) + bg
    u = jnp.dot(xb, wu, preferred_element_type=jnp.float32) + bu
    h = g * jax.nn.sigmoid(g) * u
    y = jnp.dot(h, wd, preferred_element_type=jnp.float32) + bd
    return y * rw


def _ffn_kernel(eid_s, valid_s, xs_ref, weg_ref, weu_ref, wed_ref,
                beg_ref, beu_ref, bed_ref, rw_ref, ye_ref):
    i = pl.program_id(0)

    @pl.when(valid_s[i] == 1)
    def _():
        rw = rw_ref[:, :1]
        ye_ref[...] = _ffn_body(xs_ref[...], weg_ref[0], beg_ref[0],
                                weu_ref[0], beu_ref[0], wed_ref[0],
                                bed_ref[0], rw)


def _shared_ffn_kernel(x_ref, wsg_ref, wsu_ref, wsd_ref, bsg_ref, bsu_ref,
                       bsd_ref, ysh_ref):
    ones = jnp.ones((TILE, 1), jnp.float32)
    ysh_ref[...] = _ffn_body(x_ref[...], wsg_ref[...], bsg_ref[...],
                             wsu_ref[...], bsu_ref[...], wsd_ref[...],
                             bsd_ref[...], ones)


def _run_shared_ffn(x, Wsg, bsg, Wsu, bsu, Wsd, bsd):
    return pl.pallas_call(
        _shared_ffn_kernel,
        grid=(T // TILE,),
        in_specs=[
            pl.BlockSpec((TILE, D), lambda i: (i, 0)),
            pl.BlockSpec((D, I), lambda i: (0, 0)),
            pl.BlockSpec((D, I), lambda i: (0, 0)),
            pl.BlockSpec((I, D), lambda i: (0, 0)),
            pl.BlockSpec((1, I), lambda i: (0, 0)),
            pl.BlockSpec((1, I), lambda i: (0, 0)),
            pl.BlockSpec((1, D), lambda i: (0, 0)),
        ],
        out_specs=pl.BlockSpec((TILE, D), lambda i: (i, 0)),
        out_shape=jax.ShapeDtypeStruct((T, D), jnp.float32),
    )(x, Wsg, Wsu, Wsd, bsg.reshape(1, I), bsu.reshape(1, I),
      bsd.reshape(1, D))


def _run_ffn(xs, row_w, tile_eid, tile_valid,
             Weg, beg, Weu, beu, Wed, bed):
    def emap(i, eid, val):
        return (eid[i], 0, 0)

    grid_spec = pltpu.PrefetchScalarGridSpec(
        num_scalar_prefetch=2,
        grid=(NT_R,),
        in_specs=[
            pl.BlockSpec((TILE, D), lambda i, eid, val: (i, 0)),
            pl.BlockSpec((1, D, I), emap),
            pl.BlockSpec((1, D, I), emap),
            pl.BlockSpec((1, I, D), emap),
            pl.BlockSpec((1, 1, I), emap),
            pl.BlockSpec((1, 1, I), emap),
            pl.BlockSpec((1, 1, D), emap),
            pl.BlockSpec((TILE, 128), lambda i, eid, val: (i, 0)),
        ],
        out_specs=pl.BlockSpec((TILE, D), lambda i, eid, val: (i, 0)),
    )
    return pl.pallas_call(
        _ffn_kernel,
        grid_spec=grid_spec,
        out_shape=jax.ShapeDtypeStruct((NR_R, D), jnp.float32),
    )(tile_eid, tile_valid, xs, Weg, Weu, Wed,
      beg.reshape(E, 1, I), beu.reshape(E, 1, I), bed.reshape(E, 1, D),
      row_w)


_NW = 32


def _sc_mesh():
    return plsc.VectorSubcoreMesh(core_axis_name="c", subcore_axis_name="s")


def _wid():
    return lax.axis_index("s") * 2 + lax.axis_index("c")


def _run_dispatch(x, pos0, pos1, w0r, w1r):
    toks_per_w = T // _NW

    @functools.partial(
        pl.kernel,
        out_type=[
            jax.ShapeDtypeStruct((NR_R, D), jnp.float32),
            jax.ShapeDtypeStruct((NR_R, 128), jnp.float32),
        ],
        mesh=_sc_mesh(),
        scratch_types=[
            pltpu.VMEM((toks_per_w, D), jnp.float32),
            pltpu.VMEM((toks_per_w, 128), jnp.float32),
            pltpu.VMEM((toks_per_w,), jnp.int32),
            pltpu.VMEM((toks_per_w,), jnp.int32),
            pltpu.SemaphoreType.DMA,
        ],
    )
    def k(x_hbm, p0_hbm, p1_hbm, w0_hbm, w1_hbm, xs_hbm, rw_hbm,
          xrows_v, wrep_v, i0_v, i1_v, sem):
        tb = _wid() * toks_per_w
        sl = pl.ds(tb, toks_per_w)
        pltpu.sync_copy(p0_hbm.at[sl], i0_v)
        pltpu.sync_copy(p1_hbm.at[sl], i1_v)

        pltpu.sync_copy(x_hbm.at[sl], xrows_v)
        pltpu.async_copy(xrows_v, xs_hbm.at[i0_v], sem).wait()
        pltpu.async_copy(xrows_v, xs_hbm.at[i1_v], sem).wait()

        pltpu.sync_copy(w0_hbm.at[sl], wrep_v)
        pltpu.async_copy(wrep_v, rw_hbm.at[i0_v], sem).wait()
        pltpu.sync_copy(w1_hbm.at[sl], wrep_v)
        pltpu.async_copy(wrep_v, rw_hbm.at[i1_v], sem).wait()

    return k(x, pos0, pos1, w0r, w1r)


def _run_combine_gather(ye, pos0, pos1):
    toks_per_w = T // _NW
    CH = 16

    @functools.partial(
        pl.kernel,
        out_type=jax.ShapeDtypeStruct((2, T, D), jnp.float32),
        mesh=_sc_mesh(),
        scratch_types=[
            pltpu.VMEM((CH,), jnp.int32),
            pltpu.VMEM((CH,), jnp.int32),
            pltpu.VMEM((CH, D), jnp.float32),
            pltpu.VMEM((CH, D), jnp.float32),
            pltpu.SemaphoreType.DMA,
        ],
    )
    def k(ye_hbm, p0_hbm, p1_hbm, rr_hbm, i0_v, i1_v, r0_v, r1_v, sem):
        base = _wid() * toks_per_w

        def body(c, carry):
            tb = base + c * CH
            pltpu.sync_copy(p0_hbm.at[pl.ds(tb, CH)], i0_v)
            pltpu.sync_copy(p1_hbm.at[pl.ds(tb, CH)], i1_v)
            d0 = pltpu.async_copy(ye_hbm.at[i0_v], r0_v, sem)
            d1 = pltpu.async_copy(ye_hbm.at[i1_v], r1_v, sem)
            d0.wait()
            pltpu.sync_copy(r0_v, rr_hbm.at[0, pl.ds(tb, CH)])
            d1.wait()
            pltpu.sync_copy(r1_v, rr_hbm.at[1, pl.ds(tb, CH)])
            return carry

        lax.fori_loop(0, toks_per_w // CH, body, 0)

    return k(ye, pos0, pos1)


def _add3_kernel(r0_ref, r1_ref, rs_ref, out_ref):
    out_ref[...] = r0_ref[0] + r1_ref[0] + rs_ref[...]


def _run_add3(rr, ysh):
    return pl.pallas_call(
        _add3_kernel,
        grid=(T // TILE,),
        in_specs=[
            pl.BlockSpec((1, TILE, D), lambda i: (0, i, 0)),
            pl.BlockSpec((1, TILE, D), lambda i: (1, i, 0)),
            pl.BlockSpec((TILE, D), lambda i: (i, 0)),
        ],
        out_specs=pl.BlockSpec((TILE, D), lambda i: (i, 0)),
        out_shape=jax.ShapeDtypeStruct((T, D), jnp.float32),
    )(rr, rr, ysh)


def kernel(hidden_states, Wr, br, Wsg, bsg, Wsu, bsu, Wsd, bsd, Weg, beg, Weu, beu, Wed, bed):
    x = hidden_states
    pos0c, pos1c, w0r, w1r, eidc, validc = _run_router(x, Wr, br)
    pos0, pos1 = pos0c[:, 0], pos1c[:, 0]
    tile_eid, tile_valid = eidc[:, 0], validc[:, 0]

    xs, rw = _run_dispatch(x, pos0, pos1, w0r, w1r)
    ysh = _run_shared_ffn(x, Wsg, bsg, Wsu, bsu, Wsd, bsd)
    ye = _run_ffn(xs, rw, tile_eid, tile_valid,
                  Weg, beg, Weu, beu, Wed, bed)

    rr = _run_combine_gather(ye, pos0, pos1)
    out = _run_add3(rr, ysh)
    return out
